# Initial kernel scaffold; baseline (speedup 1.0000x reference)
import jax
import jax.numpy as jnp
from jax.experimental import pallas as pl


def kernel(x, W, b, edge_index):
    raise NotImplementedError("write your pallas kernel here")


# speedup vs baseline: 24.2094x; 24.2094x over previous
import functools

import jax
import jax.numpy as jnp
from jax import lax
from jax.experimental import pallas as pl
from jax.experimental.pallas import tpu as pltpu
from jax.experimental.pallas import tpu_sc as plsc

N = 10000
E = 320000
D = 128

NC = 2
NS = 16
NW = NC * NS
EPW = E // NW
CB = 80
CH = EPW // CB
RPT = 624
TAIL = N - NS * RPT
L = 16

_mesh = plsc.VectorSubcoreMesh(core_axis_name="c", subcore_axis_name="s")
_sc_params = pltpu.CompilerParams(needs_layout_passes=False)


@functools.partial(
    pl.kernel,
    out_type=jax.ShapeDtypeStruct((NW, 1, N), jnp.float32),
    mesh=_mesh,
    scratch_types=[
        pltpu.VMEM((1, EPW), jnp.int32),
        pltpu.VMEM((1, N), jnp.float32),
    ],
    compiler_params=_sc_params,
)
def _deg_kernel(dst_hbm, deg_out, dst_v, deg_v):
    c = lax.axis_index("c")
    s = lax.axis_index("s")
    wid = s * NC + c
    pltpu.sync_copy(dst_hbm.at[wid], dst_v)

    zeros = jnp.zeros((L,), jnp.float32)

    def zero_body(i, carry):
        deg_v[0, pl.ds(i * L, L)] = zeros
        return carry

    lax.fori_loop(0, N // L, zero_body, 0)

    ones = jnp.ones((L,), jnp.float32)
    zi = jnp.zeros((L,), jnp.int32)

    def count_body(i, carry):
        idx = dst_v[0, pl.ds(i * L, L)]
        plsc.addupdate_scatter(deg_v, [zi, idx], ones)
        return carry

    lax.fori_loop(0, EPW // L, count_body, 0)
    pltpu.sync_copy(deg_v, deg_out.at[wid])


def _mm_body(degpt_ref, x_ref, w_ref, h2_ref):
    deg = jnp.sum(degpt_ref[...], axis=1, keepdims=True)
    a = lax.rsqrt(jnp.maximum(deg, 1.0))
    h = jnp.dot(x_ref[...], w_ref[...], preferred_element_type=jnp.float32)
    h2_ref[...] = h * a


def _mm(deg_pt, x, W):
    return pl.pallas_call(
        _mm_body,
        out_shape=jax.ShapeDtypeStruct((N, D), jnp.float32),
    )(deg_pt, x, W)


@functools.partial(
    pl.kernel,
    out_type=jax.ShapeDtypeStruct((NC, N, D), jnp.float32),
    mesh=_mesh,
    scratch_types=[
        pltpu.VMEM((CH, CB), jnp.int32),
        pltpu.VMEM((CH, CB), jnp.int32),
        pltpu.VMEM((CB, D), jnp.float32),
        pltpu.VMEM((CB, D), jnp.float32),
        pltpu.VMEM_SHARED((N, D), jnp.float32),
        pltpu.SemaphoreType.DMA,
        pltpu.SemaphoreType.DMA,
    ],
    compiler_params=_sc_params,
)
def _scatter_kernel(src_hbm, dst_hbm, h2_hbm, z_hbm, acc_out,
                    src_v, dst_v, rows0, rows1, acc_sh, sem0, sem1):
    c = lax.axis_index("c")
    s = lax.axis_index("s")
    wid = s * NC + c

    pltpu.sync_copy(src_hbm.at[wid], src_v)
    pltpu.sync_copy(dst_hbm.at[wid], dst_v)
    pltpu.sync_copy(z_hbm.at[pl.ds(0, RPT)], acc_sh.at[pl.ds(s * RPT, RPT)])

    @pl.when(s == NS - 1)
    def _():
        pltpu.sync_copy(z_hbm.at[pl.ds(0, TAIL)],
                        acc_sh.at[pl.ds(NS * RPT, TAIL)])

    plsc.subcore_barrier()

    def body(j, carry):
        cp = pltpu.async_copy(h2_hbm.at[src_v.at[j]], rows0, sem0)
        cp.wait()
        pltpu.sync_copy(rows0, acc_sh.at[dst_v.at[j]], add=True)
        return carry

    lax.fori_loop(0, CH, body, 0)

    plsc.subcore_barrier()
    pltpu.sync_copy(acc_sh.at[pl.ds(s * RPT, RPT)],
                    acc_out.at[c, pl.ds(s * RPT, RPT)])

    @pl.when(s == NS - 1)
    def _():
        pltpu.sync_copy(acc_sh.at[pl.ds(NS * RPT, TAIL)],
                        acc_out.at[c, pl.ds(NS * RPT, TAIL)])


def _fin_body(degpt_ref, acc_ref, b_ref, out_ref):
    deg = jnp.sum(degpt_ref[...], axis=1, keepdims=True)
    a = lax.rsqrt(jnp.maximum(deg, 1.0))
    acc = acc_ref[0] + acc_ref[1]
    out_ref[...] = acc * a + b_ref[...]


def _fin(deg_pt, acc, b2):
    return pl.pallas_call(
        _fin_body,
        out_shape=jax.ShapeDtypeStruct((N, D), jnp.float32),
    )(deg_pt, acc, b2)


@jax.jit
def kernel(x, W, b, edge_index):
    src = edge_index[0].reshape(NW, CH, CB)
    dst = edge_index[1].reshape(NW, CH, CB)
    dst_flat = edge_index[1].reshape(NW, 1, EPW)

    deg_p = _deg_kernel(dst_flat)
    deg_pt = deg_p.reshape(NW, N).T
    h2 = _mm(deg_pt, x, W)
    z = jnp.zeros((RPT, D), jnp.float32)
    acc = _scatter_kernel(src, dst, h2, z)
    return _fin(deg_pt, acc, b.reshape(1, D))
